# baseline (speedup 1.0000x reference)
import functools

import jax
import jax.numpy as jnp
from jax import lax
from jax.experimental import pallas as pl
from jax.experimental.pallas import tpu as pltpu
from jax.experimental.pallas import tpu_sc as plsc

_LANES = 16


def _seq_chunks(S):
    chunks = []
    off = 0
    while off < S:
        c = min(128, S - off)
        chunks.append((off, c))
        off += c
    return chunks


def _pooled_sc(indices, emb_table):
    B, S = indices.shape
    V, D = emb_table.shape
    info = plsc.get_sparse_core_info()
    NC, NS = info.num_cores, info.num_subcores
    NW = NC * NS
    assert B % NW == 0
    rows_per_w = B // NW
    chunks = _seq_chunks(S)
    nd = D // _LANES

    mesh = plsc.VectorSubcoreMesh(core_axis_name="c", subcore_axis_name="s")

    @functools.partial(
        pl.kernel,
        mesh=mesh,
        out_type=jax.ShapeDtypeStruct((B, D), jnp.float32),
        scratch_types=[
            pltpu.VMEM((rows_per_w, S), jnp.int32),
            pltpu.VMEM((S, D), jnp.float32),
            pltpu.VMEM((rows_per_w, D), jnp.float32),
            pltpu.SemaphoreType.DMA,
        ],
        compiler_params=pltpu.CompilerParams(use_tc_tiling_on_sc=False),
    )
    def k(idx_hbm, table_hbm, out_hbm, idx_v, buf, pooled_v, sem):
        wid = lax.axis_index("s") * NC + lax.axis_index("c")
        base = wid * rows_per_w
        pltpu.sync_copy(idx_hbm.at[pl.ds(base, rows_per_w)], idx_v)

        def row(i, carry):
            for off, c in chunks:
                pltpu.async_copy(
                    table_hbm.at[idx_v.at[i, pl.ds(off, c)]],
                    buf.at[pl.ds(off, c)],
                    sem,
                ).wait()
            accs = tuple(buf[0, pl.ds(d * _LANES, _LANES)] for d in range(nd))

            def red(j, accs):
                return tuple(
                    jnp.maximum(a, buf[j, pl.ds(d * _LANES, _LANES)])
                    for d, a in enumerate(accs)
                )

            accs = lax.fori_loop(1, S, red, accs)
            for d, a in enumerate(accs):
                pooled_v[i, pl.ds(d * _LANES, _LANES)] = a
            return carry

        lax.fori_loop(0, rows_per_w, row, 0)
        pltpu.sync_copy(pooled_v, out_hbm.at[pl.ds(base, rows_per_w)])

    return k(indices, emb_table)


def _linear_tc(pooled, W, b):
    B, D = pooled.shape
    O = W.shape[0]
    blk = 512

    def mm(p_ref, w_ref, b_ref, o_ref):
        o_ref[...] = (
            lax.dot_general(
                p_ref[...],
                w_ref[...],
                dimension_numbers=(((1,), (1,)), ((), ())),
                preferred_element_type=jnp.float32,
            )
            + b_ref[...]
        )

    return pl.pallas_call(
        mm,
        grid=(B // blk,),
        in_specs=[
            pl.BlockSpec((blk, D), lambda i: (i, 0)),
            pl.BlockSpec((O, D), lambda i: (0, 0)),
            pl.BlockSpec((1, O), lambda i: (0, 0)),
        ],
        out_specs=pl.BlockSpec((blk, O), lambda i: (i, 0)),
        out_shape=jax.ShapeDtypeStruct((B, O), jnp.float32),
    )(pooled, W, b.reshape(1, O))


def kernel(indices, emb_table, W, b):
    pooled = _pooled_sc(indices, emb_table)
    return _linear_tc(pooled, W, b)


# speedup vs baseline: 1.2683x; 1.2683x over previous
import functools

import jax
import jax.numpy as jnp
from jax import lax
from jax.experimental import pallas as pl
from jax.experimental.pallas import tpu as pltpu
from jax.experimental.pallas import tpu_sc as plsc

_LANES = 16


def _seq_chunks(S):
    chunks = []
    off = 0
    while off < S:
        c = min(128, S - off)
        chunks.append((off, c))
        off += c
    return chunks


def _pooled_sc(indices, emb_table):
    B, S = indices.shape
    V, D = emb_table.shape
    info = plsc.get_sparse_core_info()
    NC, NS = info.num_cores, info.num_subcores
    NW = NC * NS
    assert B % NW == 0
    rows_per_w = B // NW
    chunks = _seq_chunks(S)
    nd = D // _LANES

    mesh = plsc.VectorSubcoreMesh(core_axis_name="c", subcore_axis_name="s")

    assert S % 4 == 0
    nbuf = 2
    assert rows_per_w % nbuf == 0

    @functools.partial(
        pl.kernel,
        mesh=mesh,
        out_type=jax.ShapeDtypeStruct((B, D), jnp.float32),
        scratch_types=[
            pltpu.VMEM((rows_per_w, S), jnp.int32),
            pltpu.VMEM((nbuf, S, D), jnp.float32),
            pltpu.VMEM((rows_per_w, D), jnp.float32),
            pltpu.SemaphoreType.DMA,
            pltpu.SemaphoreType.DMA,
        ],
        compiler_params=pltpu.CompilerParams(use_tc_tiling_on_sc=False),
    )
    def k(idx_hbm, table_hbm, out_hbm, idx_v, buf, pooled_v, sem0, sem1):
        wid = lax.axis_index("s") * NC + lax.axis_index("c")
        base = wid * rows_per_w
        sems = (sem0, sem1)
        pltpu.sync_copy(idx_hbm.at[pl.ds(base, rows_per_w)], idx_v)

        def issue(i, slot, sem):
            for off, c in chunks:
                pltpu.async_copy(
                    table_hbm.at[idx_v.at[i, pl.ds(off, c)]],
                    buf.at[slot, pl.ds(off, c)],
                    sem,
                )

        def wait(i, slot, sem):
            for off, c in chunks:
                pltpu.make_async_copy(
                    table_hbm.at[idx_v.at[i, pl.ds(off, c)]],
                    buf.at[slot, pl.ds(off, c)],
                    sem,
                ).wait()

        issue(0, 0, sems[0])

        def outer(t, carry):
            for s in range(nbuf):
                i = t * nbuf + s
                ns = (s + 1) % nbuf

                @pl.when(i + 1 < rows_per_w)
                def _():
                    issue(i + 1, ns, sems[ns])

                wait(i, s, sems[s])

                accs = tuple(
                    buf[s, 0, pl.ds(d * _LANES, _LANES)] for d in range(nd)
                )
                for r in (1, 2, 3):
                    accs = tuple(
                        jnp.maximum(a, buf[s, r, pl.ds(d * _LANES, _LANES)])
                        for d, a in enumerate(accs)
                    )

                def red4(kk, accs, s=s):
                    jbase = kk * 4
                    for r in range(4):
                        accs = tuple(
                            jnp.maximum(
                                a, buf[s, jbase + r, pl.ds(d * _LANES, _LANES)]
                            )
                            for d, a in enumerate(accs)
                        )
                    return accs

                accs = lax.fori_loop(1, S // 4, red4, accs)
                for d, a in enumerate(accs):
                    pooled_v[i, pl.ds(d * _LANES, _LANES)] = a
            return carry

        lax.fori_loop(0, rows_per_w // nbuf, outer, 0)
        pltpu.sync_copy(pooled_v, out_hbm.at[pl.ds(base, rows_per_w)])

    return k(indices, emb_table)


def _linear_tc(pooled, W, b):
    B, D = pooled.shape
    O = W.shape[0]
    blk = 512

    def mm(p_ref, w_ref, b_ref, o_ref):
        o_ref[...] = (
            lax.dot_general(
                p_ref[...],
                w_ref[...],
                dimension_numbers=(((1,), (1,)), ((), ())),
                preferred_element_type=jnp.float32,
            )
            + b_ref[...]
        )

    return pl.pallas_call(
        mm,
        grid=(B // blk,),
        in_specs=[
            pl.BlockSpec((blk, D), lambda i: (i, 0)),
            pl.BlockSpec((O, D), lambda i: (0, 0)),
            pl.BlockSpec((1, O), lambda i: (0, 0)),
        ],
        out_specs=pl.BlockSpec((blk, O), lambda i: (i, 0)),
        out_shape=jax.ShapeDtypeStruct((B, O), jnp.float32),
    )(pooled, W, b.reshape(1, O))


def kernel(indices, emb_table, W, b):
    pooled = _pooled_sc(indices, emb_table)
    return _linear_tc(pooled, W, b)
